# baseline (speedup 1.0000x reference)
import functools

import jax
import jax.numpy as jnp
from jax import lax
from jax.experimental import pallas as pl
from jax.experimental.pallas import tpu as pltpu
from jax.experimental.pallas import tpu_sc as plsc

E = 16
TOP_K = 2
D_IN = 768
D_HID = 2048
D_OUT = 768
T = 2048
BM = 352
G = 28
PROWS = G * BM
LB_WEIGHT = 0.01

_NC = 2
_NS = 16
_NW = _NC * _NS
_CH = T // _NW


def _router_body(x_ref, wg_ref, bg_ref, pos_ref, w_ref, estep_ref, srow_ref,
                 loss_ref):
    x = x_ref[...]
    logits = jnp.dot(x, wg_ref[...], preferred_element_type=jnp.float32)
    logits = logits + bg_ref[...]
    m = jnp.max(logits, axis=1, keepdims=True)
    p = jnp.exp(logits - m)
    p = p / jnp.sum(p, axis=1, keepdims=True)

    ei = lax.broadcasted_iota(jnp.int32, (T, E), 1)
    v0 = jnp.max(p, axis=1, keepdims=True)
    i0 = jnp.min(jnp.where(p == v0, ei, E), axis=1, keepdims=True)
    c0 = ei == i0
    p2 = jnp.where(c0, -jnp.inf, p)
    v1 = jnp.max(p2, axis=1, keepdims=True)
    i1 = jnp.min(jnp.where(p2 == v1, ei, E), axis=1, keepdims=True)
    c1 = ei == i1
    s01 = v0 + v1
    w_ref[:, 0:1] = v0 / s01
    w_ref[:, 1:2] = v1 / s01

    meanp = jnp.sum(p, axis=0, keepdims=True) / T
    cnt = c0.astype(jnp.float32) + c1.astype(jnp.float32)
    counts = jnp.sum(cnt, axis=0, keepdims=True)
    frac = counts / (T * TOP_K)
    loss_ref[...] = LB_WEIGHT * E * jnp.sum(frac * meanp, axis=1,
                                            keepdims=True)

    c = cnt
    off = 1
    while off < T:
        c = c + jnp.concatenate(
            [jnp.zeros((off, E), jnp.float32), c[: T - off]], axis=0)
        off *= 2
    cume = c - cnt

    blocks = jnp.floor((counts + (BM - 1)) / BM)
    eio = lax.broadcasted_iota(jnp.int32, (E, E), 0)
    ejo = lax.broadcasted_iota(jnp.int32, (E, E), 1)
    strict_lt = (eio < ejo).astype(jnp.float32)
    cumexcl = jnp.dot(blocks, strict_lt,
                      preferred_element_type=jnp.float32)
    cuminc = cumexcl + blocks
    total_i = jnp.sum(blocks).astype(jnp.int32)
    padded_off = cumexcl * BM

    pos0 = jnp.sum(jnp.where(c0, cume + padded_off, 0.0), axis=1,
                   keepdims=True)
    pos1 = jnp.sum(jnp.where(c1, cume + padded_off, 0.0), axis=1,
                   keepdims=True)
    pos_ref[:, 0:1] = pos0.astype(jnp.int32)
    pos_ref[:, 1:2] = pos1.astype(jnp.int32)

    s_io = lax.broadcasted_iota(jnp.int32, (G, E), 0)
    s_cl = jnp.minimum(s_io, total_i - 1)
    cuminc_i = cuminc.astype(jnp.int32)
    estep_ref[...] = jnp.sum((s_cl >= cuminc_i).astype(jnp.int32), axis=1,
                             keepdims=True)
    srow_ref[...] = jnp.minimum(
        lax.broadcasted_iota(jnp.int32, (G, 1), 0), total_i - 1)


def _router(x, Wg, bg):
    return pl.pallas_call(
        _router_body,
        out_shape=(
            jax.ShapeDtypeStruct((T, 2), jnp.int32),
            jax.ShapeDtypeStruct((T, 2), jnp.float32),
            jax.ShapeDtypeStruct((G, 1), jnp.int32),
            jax.ShapeDtypeStruct((G, 1), jnp.int32),
            jax.ShapeDtypeStruct((1, 1), jnp.float32),
        ),
    )(x, Wg, bg)


def _dispatch_body(x_hbm, pos0_hbm, pos1_hbm, xg_hbm, idx0, idx1, rows_v, sem):
    wid = lax.axis_index("s") * _NC + lax.axis_index("c")
    base = wid * _CH
    pltpu.sync_copy(pos0_hbm.at[pl.ds(base, _CH)], idx0)
    pltpu.sync_copy(pos1_hbm.at[pl.ds(base, _CH)], idx1)
    pltpu.sync_copy(x_hbm.at[pl.ds(base, _CH)], rows_v)
    c0 = pltpu.async_copy(rows_v, xg_hbm.at[idx0], sem)
    c1 = pltpu.async_copy(rows_v, xg_hbm.at[idx1], sem)
    c0.wait()
    c1.wait()


@functools.lru_cache(maxsize=None)
def _sc_mesh():
    return plsc.VectorSubcoreMesh(
        core_axis_name="c", subcore_axis_name="s",
        num_cores=_NC, num_subcores=_NS)


@functools.lru_cache(maxsize=None)
def _dispatch_kernel():
    return pl.kernel(
        _dispatch_body,
        mesh=_sc_mesh(),
        out_type=jax.ShapeDtypeStruct((PROWS, D_IN), jnp.float32),
        scratch_types=[
            pltpu.VMEM((_CH,), jnp.int32),
            pltpu.VMEM((_CH,), jnp.int32),
            pltpu.VMEM((_CH, D_IN), jnp.float32),
            pltpu.SemaphoreType.DMA,
        ],
    )


def _gmm_body(estep_s, srow_s, xg_ref, w1_ref, b1_ref, w2_ref, b2_ref,
              pos0_ref, pos1_ref, wt0_ref, wt1_ref, y_ref):
    s = pl.program_id(0)

    @pl.when(srow_s[s] == s)
    def _():
        xb = xg_ref[...]
        h = jnp.dot(xb, w1_ref[0], preferred_element_type=jnp.float32)
        h = jnp.maximum(h + b1_ref[0], 0.0)
        y = jnp.dot(h, w2_ref[0], preferred_element_type=jnp.float32)
        rowid = s * BM + lax.broadcasted_iota(jnp.int32, (BM, 1), 0)
        m0 = (pos0_ref[...] == rowid).astype(jnp.float32)
        m1 = (pos1_ref[...] == rowid).astype(jnp.float32)
        ws = (jnp.dot(m0, wt0_ref[...], preferred_element_type=jnp.float32)
              + jnp.dot(m1, wt1_ref[...], preferred_element_type=jnp.float32))
        y_ref[...] = (y + b2_ref[0]) * ws


def _gmm(estep, srow, xg, W1, b1, W2, b2, pos0, pos1, wt0, wt1):
    grid_spec = pltpu.PrefetchScalarGridSpec(
        num_scalar_prefetch=2,
        grid=(G,),
        in_specs=[
            pl.BlockSpec((BM, D_IN), lambda s, es, sr: (sr[s], 0)),
            pl.BlockSpec((1, D_IN, D_HID), lambda s, es, sr: (es[s], 0, 0)),
            pl.BlockSpec((1, 1, D_HID), lambda s, es, sr: (es[s], 0, 0)),
            pl.BlockSpec((1, D_HID, D_OUT), lambda s, es, sr: (es[s], 0, 0)),
            pl.BlockSpec((1, 1, D_OUT), lambda s, es, sr: (es[s], 0, 0)),
            pl.BlockSpec((1, T), lambda s, es, sr: (0, 0)),
            pl.BlockSpec((1, T), lambda s, es, sr: (0, 0)),
            pl.BlockSpec((T, 1), lambda s, es, sr: (0, 0)),
            pl.BlockSpec((T, 1), lambda s, es, sr: (0, 0)),
        ],
        out_specs=pl.BlockSpec((BM, D_OUT), lambda s, es, sr: (sr[s], 0)),
    )
    return pl.pallas_call(
        _gmm_body,
        grid_spec=grid_spec,
        out_shape=jax.ShapeDtypeStruct((PROWS, D_OUT), jnp.float32),
    )(estep, srow, xg, W1, b1, W2, b2, pos0.reshape(1, T), pos1.reshape(1, T),
      wt0.reshape(T, 1), wt1.reshape(T, 1))


_NCHUNK = 4
_CC = _CH // _NCHUNK


def _combine_body(y_hbm, pos0_hbm, pos1_hbm, out_hbm, idx0, idx1, buf0, buf1,
                  sem0, sem1):
    wid = lax.axis_index("s") * _NC + lax.axis_index("c")
    base = wid * _CH
    pltpu.sync_copy(pos0_hbm.at[pl.ds(base, _CH)], idx0)
    pltpu.sync_copy(pos1_hbm.at[pl.ds(base, _CH)], idx1)
    cps = []
    for c in range(_NCHUNK):
        cs = pl.ds(c * _CC, _CC)
        cps.append((pltpu.async_copy(y_hbm.at[idx0.at[cs]], buf0.at[cs],
                                     sem0.at[c]),
                    pltpu.async_copy(y_hbm.at[idx1.at[cs]], buf1.at[cs],
                                     sem1.at[c])))
    for c in range(_NCHUNK):
        cp0, cp1 = cps[c]
        cp0.wait()
        cp1.wait()

        def row_add(i, carry):
            for j in range(D_OUT // 16):
                sl = pl.ds(j * 16, 16)
                buf0[i, sl] = buf0[i, sl] + buf1[i, sl]
            return carry

        lax.fori_loop(c * _CC, (c + 1) * _CC, row_add, 0)
        pltpu.sync_copy(buf0.at[pl.ds(c * _CC, _CC)],
                        out_hbm.at[pl.ds(base + c * _CC, _CC)])


@functools.lru_cache(maxsize=None)
def _combine_kernel():
    return pl.kernel(
        _combine_body,
        mesh=_sc_mesh(),
        out_type=jax.ShapeDtypeStruct((T, D_OUT), jnp.float32),
        scratch_types=[
            pltpu.VMEM((_CH,), jnp.int32),
            pltpu.VMEM((_CH,), jnp.int32),
            pltpu.VMEM((_CH, D_OUT), jnp.float32),
            pltpu.VMEM((_CH, D_OUT), jnp.float32),
            pltpu.SemaphoreType.DMA((_NCHUNK,)),
            pltpu.SemaphoreType.DMA((_NCHUNK,)),
        ],
    )


def kernel(input_tensor, Wg, bg, W1, b1, W2, b2):
    x = input_tensor.reshape(T, D_IN)
    pos, w, estep, srow, loss = _router(x, Wg, bg.reshape(1, E))
    pos0 = pos[:, 0]
    pos1 = pos[:, 1]
    xg = _dispatch_kernel()(x, pos0, pos1)
    y = _gmm(estep.reshape(G), srow.reshape(G), xg, W1,
             b1.reshape(E, 1, D_HID), W2, b2.reshape(E, 1, D_OUT),
             pos0, pos1, w[:, 0], w[:, 1])
    out = _combine_kernel()(y, pos0, pos1)
    return out.reshape(1, T, D_OUT), loss[0, 0]


# speedup vs baseline: 1.0206x; 1.0206x over previous
import functools

import jax
import jax.numpy as jnp
from jax import lax
from jax.experimental import pallas as pl
from jax.experimental.pallas import tpu as pltpu
from jax.experimental.pallas import tpu_sc as plsc

E = 16
TOP_K = 2
D_IN = 768
D_HID = 2048
D_OUT = 768
T = 2048
BM = 320
G = 29
PROWS = G * BM
LB_WEIGHT = 0.01

_NC = 2
_NS = 16
_NW = _NC * _NS
_CH = T // _NW


def _router_body(x_ref, wg_ref, bg_ref, pos_ref, w_ref, estep_ref, srow_ref,
                 loss_ref):
    x = x_ref[...]
    logits = jnp.dot(x, wg_ref[...], preferred_element_type=jnp.float32)
    logits = logits + bg_ref[...]
    m = jnp.max(logits, axis=1, keepdims=True)
    p = jnp.exp(logits - m)
    p = p / jnp.sum(p, axis=1, keepdims=True)

    ei = lax.broadcasted_iota(jnp.int32, (T, E), 1)
    v0 = jnp.max(p, axis=1, keepdims=True)
    i0 = jnp.min(jnp.where(p == v0, ei, E), axis=1, keepdims=True)
    c0 = ei == i0
    p2 = jnp.where(c0, -jnp.inf, p)
    v1 = jnp.max(p2, axis=1, keepdims=True)
    i1 = jnp.min(jnp.where(p2 == v1, ei, E), axis=1, keepdims=True)
    c1 = ei == i1
    s01 = v0 + v1
    w_ref[:, 0:1] = v0 / s01
    w_ref[:, 1:2] = v1 / s01

    meanp = jnp.sum(p, axis=0, keepdims=True) / T
    cnt = c0.astype(jnp.float32) + c1.astype(jnp.float32)
    counts = jnp.sum(cnt, axis=0, keepdims=True)
    frac = counts / (T * TOP_K)
    loss_ref[...] = LB_WEIGHT * E * jnp.sum(frac * meanp, axis=1,
                                            keepdims=True)

    c = cnt
    off = 1
    while off < T:
        c = c + jnp.concatenate(
            [jnp.zeros((off, E), jnp.float32), c[: T - off]], axis=0)
        off *= 2
    cume = c - cnt

    blocks = jnp.floor((counts + (BM - 1)) / BM)
    eio = lax.broadcasted_iota(jnp.int32, (E, E), 0)
    ejo = lax.broadcasted_iota(jnp.int32, (E, E), 1)
    strict_lt = (eio < ejo).astype(jnp.float32)
    cumexcl = jnp.dot(blocks, strict_lt,
                      preferred_element_type=jnp.float32)
    cuminc = cumexcl + blocks
    total_i = jnp.sum(blocks).astype(jnp.int32)
    padded_off = cumexcl * BM

    pos0 = jnp.sum(jnp.where(c0, cume + padded_off, 0.0), axis=1,
                   keepdims=True)
    pos1 = jnp.sum(jnp.where(c1, cume + padded_off, 0.0), axis=1,
                   keepdims=True)
    pos_ref[:, 0:1] = pos0.astype(jnp.int32)
    pos_ref[:, 1:2] = pos1.astype(jnp.int32)

    s_io = lax.broadcasted_iota(jnp.int32, (G, E), 0)
    s_cl = jnp.minimum(s_io, total_i - 1)
    cuminc_i = cuminc.astype(jnp.int32)
    estep_ref[...] = jnp.sum((s_cl >= cuminc_i).astype(jnp.int32), axis=1,
                             keepdims=True)
    srow_ref[...] = jnp.minimum(
        lax.broadcasted_iota(jnp.int32, (G, 1), 0), total_i - 1)


def _router(x, Wg, bg):
    return pl.pallas_call(
        _router_body,
        out_shape=(
            jax.ShapeDtypeStruct((T, 2), jnp.int32),
            jax.ShapeDtypeStruct((T, 2), jnp.float32),
            jax.ShapeDtypeStruct((G, 1), jnp.int32),
            jax.ShapeDtypeStruct((G, 1), jnp.int32),
            jax.ShapeDtypeStruct((1, 1), jnp.float32),
        ),
    )(x, Wg, bg)


def _dispatch_body(x_hbm, pos0_hbm, pos1_hbm, xg_hbm, idx0, idx1, rows_v, sem):
    wid = lax.axis_index("s") * _NC + lax.axis_index("c")
    base = wid * _CH
    pltpu.sync_copy(pos0_hbm.at[pl.ds(base, _CH)], idx0)
    pltpu.sync_copy(pos1_hbm.at[pl.ds(base, _CH)], idx1)
    pltpu.sync_copy(x_hbm.at[pl.ds(base, _CH)], rows_v)
    c0 = pltpu.async_copy(rows_v, xg_hbm.at[idx0], sem)
    c1 = pltpu.async_copy(rows_v, xg_hbm.at[idx1], sem)
    c0.wait()
    c1.wait()


@functools.lru_cache(maxsize=None)
def _sc_mesh():
    return plsc.VectorSubcoreMesh(
        core_axis_name="c", subcore_axis_name="s",
        num_cores=_NC, num_subcores=_NS)


@functools.lru_cache(maxsize=None)
def _dispatch_kernel():
    return pl.kernel(
        _dispatch_body,
        mesh=_sc_mesh(),
        out_type=jax.ShapeDtypeStruct((PROWS, D_IN), jnp.float32),
        scratch_types=[
            pltpu.VMEM((_CH,), jnp.int32),
            pltpu.VMEM((_CH,), jnp.int32),
            pltpu.VMEM((_CH, D_IN), jnp.float32),
            pltpu.SemaphoreType.DMA,
        ],
    )


def _gmm_body(estep_s, srow_s, xg_ref, w1_ref, b1_ref, w2_ref, b2_ref,
              pos0_ref, pos1_ref, wt0_ref, wt1_ref, y_ref):
    s = pl.program_id(0)

    @pl.when(srow_s[s] == s)
    def _():
        xb = xg_ref[...]
        h = jnp.dot(xb, w1_ref[0], preferred_element_type=jnp.float32)
        h = jnp.maximum(h + b1_ref[0], 0.0)
        y = jnp.dot(h, w2_ref[0], preferred_element_type=jnp.float32)
        rowid = s * BM + lax.broadcasted_iota(jnp.int32, (BM, 1), 0)
        m0 = (pos0_ref[...] == rowid).astype(jnp.float32)
        m1 = (pos1_ref[...] == rowid).astype(jnp.float32)
        ws = (jnp.dot(m0, wt0_ref[...], preferred_element_type=jnp.float32)
              + jnp.dot(m1, wt1_ref[...], preferred_element_type=jnp.float32))
        y_ref[...] = (y + b2_ref[0]) * ws


def _gmm(estep, srow, xg, W1, b1, W2, b2, pos0, pos1, wt0, wt1):
    grid_spec = pltpu.PrefetchScalarGridSpec(
        num_scalar_prefetch=2,
        grid=(G,),
        in_specs=[
            pl.BlockSpec((BM, D_IN), lambda s, es, sr: (sr[s], 0)),
            pl.BlockSpec((1, D_IN, D_HID), lambda s, es, sr: (es[s], 0, 0)),
            pl.BlockSpec((1, 1, D_HID), lambda s, es, sr: (es[s], 0, 0)),
            pl.BlockSpec((1, D_HID, D_OUT), lambda s, es, sr: (es[s], 0, 0)),
            pl.BlockSpec((1, 1, D_OUT), lambda s, es, sr: (es[s], 0, 0)),
            pl.BlockSpec((1, T), lambda s, es, sr: (0, 0)),
            pl.BlockSpec((1, T), lambda s, es, sr: (0, 0)),
            pl.BlockSpec((T, 1), lambda s, es, sr: (0, 0)),
            pl.BlockSpec((T, 1), lambda s, es, sr: (0, 0)),
        ],
        out_specs=pl.BlockSpec((BM, D_OUT), lambda s, es, sr: (sr[s], 0)),
    )
    return pl.pallas_call(
        _gmm_body,
        grid_spec=grid_spec,
        out_shape=jax.ShapeDtypeStruct((PROWS, D_OUT), jnp.float32),
    )(estep, srow, xg, W1, b1, W2, b2, pos0.reshape(1, T), pos1.reshape(1, T),
      wt0.reshape(T, 1), wt1.reshape(T, 1))


_NCHUNK = 4
_CC = _CH // _NCHUNK


def _combine_body(y_hbm, pos0_hbm, pos1_hbm, out_hbm, idx0, idx1, buf0, buf1,
                  sem0, sem1):
    wid = lax.axis_index("s") * _NC + lax.axis_index("c")
    base = wid * _CH
    pltpu.sync_copy(pos0_hbm.at[pl.ds(base, _CH)], idx0)
    pltpu.sync_copy(pos1_hbm.at[pl.ds(base, _CH)], idx1)
    cps = []
    for c in range(_NCHUNK):
        cs = pl.ds(c * _CC, _CC)
        cps.append((pltpu.async_copy(y_hbm.at[idx0.at[cs]], buf0.at[cs],
                                     sem0.at[c]),
                    pltpu.async_copy(y_hbm.at[idx1.at[cs]], buf1.at[cs],
                                     sem1.at[c])))
    for c in range(_NCHUNK):
        cp0, cp1 = cps[c]
        cp0.wait()
        cp1.wait()

        def row_add(i, carry):
            for j in range(D_OUT // 16):
                sl = pl.ds(j * 16, 16)
                buf0[i, sl] = buf0[i, sl] + buf1[i, sl]
            return carry

        lax.fori_loop(c * _CC, (c + 1) * _CC, row_add, 0)
        pltpu.sync_copy(buf0.at[pl.ds(c * _CC, _CC)],
                        out_hbm.at[pl.ds(base + c * _CC, _CC)])


@functools.lru_cache(maxsize=None)
def _combine_kernel():
    return pl.kernel(
        _combine_body,
        mesh=_sc_mesh(),
        out_type=jax.ShapeDtypeStruct((T, D_OUT), jnp.float32),
        scratch_types=[
            pltpu.VMEM((_CH,), jnp.int32),
            pltpu.VMEM((_CH,), jnp.int32),
            pltpu.VMEM((_CH, D_OUT), jnp.float32),
            pltpu.VMEM((_CH, D_OUT), jnp.float32),
            pltpu.SemaphoreType.DMA((_NCHUNK,)),
            pltpu.SemaphoreType.DMA((_NCHUNK,)),
        ],
    )


def kernel(input_tensor, Wg, bg, W1, b1, W2, b2):
    x = input_tensor.reshape(T, D_IN)
    pos, w, estep, srow, loss = _router(x, Wg, bg.reshape(1, E))
    pos0 = pos[:, 0]
    pos1 = pos[:, 1]
    xg = _dispatch_kernel()(x, pos0, pos1)
    y = _gmm(estep.reshape(G), srow.reshape(G), xg, W1,
             b1.reshape(E, 1, D_HID), W2, b2.reshape(E, 1, D_OUT),
             pos0, pos1, w[:, 0], w[:, 1])
    out = _combine_kernel()(y, pos0, pos1)
    return out.reshape(1, T, D_OUT), loss[0, 0]
